# baseline (speedup 1.0000x reference)
import functools

import jax
import jax.numpy as jnp
from jax import lax
from jax.experimental import pallas as pl
from jax.experimental.pallas import tpu as pltpu
from jax.experimental.pallas import tpu_sc as plsc

_VOCAB = 10
_EMB = 20
_QPR = 50
_KROW = 128


def _quad_table_body(table_ref, w_ref, b_ref, t4_ref):
    t = (
        lax.dot_general(
            table_ref[...], w_ref[...],
            dimension_numbers=(((1,), (1,)), ((), ())),
            preferred_element_type=jnp.float32,
            precision=lax.Precision.HIGHEST,
        )
        + b_ref[...]
    )
    v = _VOCAB
    left = jnp.broadcast_to(t[:, None, :], (v, v, _EMB)).reshape(v * v, _EMB)
    right = jnp.broadcast_to(t[None, :, :], (v, v, _EMB)).reshape(v * v, _EMB)
    t2 = jnp.concatenate([left, right], axis=1)
    p = v * v
    left4 = jnp.broadcast_to(t2[:, None, :], (p, p, 2 * _EMB)).reshape(p * p, 2 * _EMB)
    right4 = jnp.broadcast_to(t2[None, :, :], (p, p, 2 * _EMB)).reshape(p * p, 2 * _EMB)
    t4_ref[...] = jnp.concatenate([left4, right4], axis=1)


def _quad_table(table, W, b):
    V, E = table.shape
    return pl.pallas_call(
        _quad_table_body,
        out_shape=jax.ShapeDtypeStruct((V**4, 4 * E), jnp.float32),
    )(table, W, b.reshape(1, E))


def _keys_body(x_ref, k_ref):
    bm, L = x_ref.shape
    xf = x_ref[...].astype(jnp.float32)
    d = lax.broadcasted_iota(jnp.int32, (L, _QPR), 0)
    q = lax.broadcasted_iota(jnp.int32, (L, _QPR), 1)
    pa = jnp.where(d == 4 * q, 10.0, 0.0) + jnp.where(d == 4 * q + 1, 1.0, 0.0)
    pb = jnp.where(d == 4 * q + 2, 10.0, 0.0) + jnp.where(d == 4 * q + 3, 1.0, 0.0)
    ka = lax.dot_general(xf, pa, (((1,), (0,)), ((), ())),
                         preferred_element_type=jnp.float32,
                         precision=lax.Precision.HIGHEST)
    kb = lax.dot_general(xf, pb, (((1,), (0,)), ((), ())),
                         preferred_element_type=jnp.float32,
                         precision=lax.Precision.HIGHEST)
    k_ref[...] = ka.astype(jnp.int32) * 100 + kb.astype(jnp.int32)


def _quad_keys(x):
    B, L = x.shape
    BM = 512
    return pl.pallas_call(
        _keys_body,
        out_shape=jax.ShapeDtypeStruct((B, _QPR), jnp.int32),
        grid=(B // BM,),
        in_specs=[pl.BlockSpec((BM, L), lambda i: (i, 0))],
        out_specs=pl.BlockSpec((BM, _QPR), lambda i: (i, 0)),
    )(x)


_BQ = 200
_GROUP = 2


def _sc_gather(T4, keys):
    NB = keys.shape[0] // 2
    D = T4.shape[1]
    info = plsc.get_sparse_core_info()
    NC, NS = info.num_cores, info.num_subcores
    NW = NC * NS
    blk_per_worker = NB // NW
    n_iter = blk_per_worker // (2 * _GROUP)

    mesh = plsc.VectorSubcoreMesh(core_axis_name="c", subcore_axis_name="s")

    @functools.partial(
        pl.kernel,
        out_type=jax.ShapeDtypeStruct((NB, _BQ, D), jnp.float32),
        mesh=mesh,
        scratch_types=[
            pltpu.VMEM((2, 2 * _GROUP, _KROW), jnp.int32),
            pltpu.VMEM((2, _GROUP, _BQ, D), jnp.float32),
            pltpu.SemaphoreType.DMA,
            pltpu.SemaphoreType.DMA,
            pltpu.SemaphoreType.DMA,
            pltpu.SemaphoreType.DMA,
        ],
        compiler_params=pltpu.CompilerParams(use_tc_tiling_on_sc=False),
    )
    def k(t4_hbm, k_hbm, out_hbm, keys_v, rows_v, sga, sgb, swa, swb):
        wid = lax.axis_index("s") * NC + lax.axis_index("c")
        base = wid * blk_per_worker

        def stage_and_gather(g, b0, sem):
            pltpu.sync_copy(k_hbm.at[pl.ds(2 * b0, 2 * _GROUP)], keys_v.at[g])
            copies = []
            for blk in range(_GROUP):
                copies.append(pltpu.async_copy(
                    t4_hbm.at[keys_v.at[g, 2 * blk, pl.ds(0, _KROW)]],
                    rows_v.at[g, blk, pl.ds(0, _KROW)], sem))
                copies.append(pltpu.async_copy(
                    t4_hbm.at[keys_v.at[g, 2 * blk + 1, pl.ds(0, _BQ - _KROW)]],
                    rows_v.at[g, blk, pl.ds(_KROW, _BQ - _KROW)], sem))
            return copies

        def drain_write(g, sem):
            pltpu.make_async_copy(
                out_hbm.at[pl.ds(0, _GROUP)], rows_v.at[g], sem).wait()

        def body(s, _):
            ba = base + s * 2 * _GROUP
            bb = ba + _GROUP

            @pl.when(s > 0)
            def _():
                drain_write(0, swa)
            ga = stage_and_gather(0, ba, sga)

            @pl.when(s > 0)
            def _():
                drain_write(1, swb)
            gb = stage_and_gather(1, bb, sgb)

            for c in ga:
                c.wait()
            pltpu.async_copy(rows_v.at[0], out_hbm.at[pl.ds(ba, _GROUP)], swa)
            for c in gb:
                c.wait()
            pltpu.async_copy(rows_v.at[1], out_hbm.at[pl.ds(bb, _GROUP)], swb)
            return ()

        lax.fori_loop(0, n_iter, body, ())
        drain_write(0, swa)
        drain_write(1, swb)

    return k(T4, keys)


def kernel(x, table, W, b):
    B, L = x.shape
    T4 = _quad_table(table, W, b)
    k = _quad_keys(x)
    k = jnp.pad(k.reshape(B // 4, 4 * _QPR), ((0, 0), (0, 2 * _KROW - 4 * _QPR)))
    keys = k.reshape(B // 2, _KROW)
    out = _sc_gather(T4, keys)
    return out.reshape(B, L, _EMB)


# speedup vs baseline: 1.0025x; 1.0025x over previous
import functools

import jax
import jax.numpy as jnp
from jax import lax
from jax.experimental import pallas as pl
from jax.experimental.pallas import tpu as pltpu
from jax.experimental.pallas import tpu_sc as plsc

_VOCAB = 10
_EMB = 20
_QPR = 50
_KROW = 128


def _quad_table_body(table_ref, w_ref, b_ref, t4_ref):
    t = (
        lax.dot_general(
            table_ref[...], w_ref[...],
            dimension_numbers=(((1,), (1,)), ((), ())),
            preferred_element_type=jnp.float32,
            precision=lax.Precision.HIGHEST,
        )
        + b_ref[...]
    )
    v = _VOCAB
    left = jnp.broadcast_to(t[:, None, :], (v, v, _EMB)).reshape(v * v, _EMB)
    right = jnp.broadcast_to(t[None, :, :], (v, v, _EMB)).reshape(v * v, _EMB)
    t2 = jnp.concatenate([left, right], axis=1)
    p = v * v
    left4 = jnp.broadcast_to(t2[:, None, :], (p, p, 2 * _EMB)).reshape(p * p, 2 * _EMB)
    right4 = jnp.broadcast_to(t2[None, :, :], (p, p, 2 * _EMB)).reshape(p * p, 2 * _EMB)
    t4_ref[...] = jnp.concatenate([left4, right4], axis=1)


def _quad_table(table, W, b):
    V, E = table.shape
    return pl.pallas_call(
        _quad_table_body,
        out_shape=jax.ShapeDtypeStruct((V**4, 4 * E), jnp.float32),
    )(table, W, b.reshape(1, E))


def _keys_body(x_ref, k_ref):
    bm, L = x_ref.shape
    xf = x_ref[...].astype(jnp.float32)
    d = lax.broadcasted_iota(jnp.int32, (L, _QPR), 0)
    q = lax.broadcasted_iota(jnp.int32, (L, _QPR), 1)
    pa = jnp.where(d == 4 * q, 10.0, 0.0) + jnp.where(d == 4 * q + 1, 1.0, 0.0)
    pb = jnp.where(d == 4 * q + 2, 10.0, 0.0) + jnp.where(d == 4 * q + 3, 1.0, 0.0)
    ka = lax.dot_general(xf, pa, (((1,), (0,)), ((), ())),
                         preferred_element_type=jnp.float32,
                         precision=lax.Precision.HIGHEST)
    kb = lax.dot_general(xf, pb, (((1,), (0,)), ((), ())),
                         preferred_element_type=jnp.float32,
                         precision=lax.Precision.HIGHEST)
    k = ka.astype(jnp.int32) * 100 + kb.astype(jnp.int32)
    k_ref[...] = jnp.concatenate(
        [k, jnp.zeros((bm, _KROW - _QPR), jnp.int32)], axis=1)


def _quad_keys(x):
    B, L = x.shape
    BM = 512
    return pl.pallas_call(
        _keys_body,
        out_shape=jax.ShapeDtypeStruct((B, _KROW), jnp.int32),
        grid=(B // BM,),
        in_specs=[pl.BlockSpec((BM, L), lambda i: (i, 0))],
        out_specs=pl.BlockSpec((BM, _KROW), lambda i: (i, 0)),
    )(x)


_BQ = 200
_GROUP = 2


_GROWS = 4 * _GROUP
_GQ = _GROWS * _QPR
_CHUNKS = ((0, 128), (128, 72), (200, 128), (328, 72))


def _sc_gather(T4, keys):
    B = keys.shape[0] // _KROW
    D = T4.shape[1]
    NB = B // 4
    info = plsc.get_sparse_core_info()
    NC, NS = info.num_cores, info.num_subcores
    NW = NC * NS
    rows_per_worker = B // NW
    n_iter = rows_per_worker // (2 * _GROWS)

    mesh = plsc.VectorSubcoreMesh(core_axis_name="c", subcore_axis_name="s")

    @functools.partial(
        pl.kernel,
        out_type=jax.ShapeDtypeStruct((NB, _BQ, D), jnp.float32),
        mesh=mesh,
        scratch_types=[
            pltpu.VMEM((_GROWS * _KROW,), jnp.int32),
            pltpu.VMEM((_GROWS * _KROW,), jnp.int32),
            pltpu.VMEM((512,), jnp.int32),
            pltpu.VMEM((512,), jnp.int32),
            pltpu.VMEM((2, _GROUP, _BQ, D), jnp.float32),
            pltpu.SemaphoreType.DMA,
            pltpu.SemaphoreType.DMA,
            pltpu.SemaphoreType.DMA,
            pltpu.SemaphoreType.DMA,
        ],
        compiler_params=pltpu.CompilerParams(
            use_tc_tiling_on_sc=False, needs_layout_passes=False),
    )
    def k(t4_hbm, k_hbm, out_hbm, ka_v, kb_v, kfa_v, kfb_v, rows_v,
          sga, sgb, swa, swb):
        wid = lax.axis_index("s") * NC + lax.axis_index("c")
        base = wid * rows_per_worker

        def stage_and_gather(g, r0, sem):
            keys_v = (ka_v, kb_v)[g]
            kflat_v = (kfa_v, kfb_v)[g]
            pltpu.sync_copy(k_hbm.at[pl.ds(r0 * _KROW, _GROWS * _KROW)],
                            keys_v)
            for c in range(_GQ // 16):
                p = c * 16 + lax.iota(jnp.int32, 16)
                j = lax.shift_right_logical(p * 1311, 16)
                vals = plsc.load_gather(keys_v, [p + j * (_KROW - _QPR)])
                kflat_v[pl.ds(c * 16, 16)] = vals
            return [
                pltpu.async_copy(
                    t4_hbm.at[kflat_v.at[pl.ds(off, n)]],
                    rows_v.at[g, off // _BQ, pl.ds(off % _BQ, n)], sem)
                for off, n in _CHUNKS
            ]

        def drain_write(g, sem):
            pltpu.make_async_copy(
                out_hbm.at[pl.ds(0, _GROUP)], rows_v.at[g], sem).wait()

        def body(s, _):
            ra = base + s * 2 * _GROWS
            rb = ra + _GROWS

            @pl.when(s > 0)
            def _():
                drain_write(0, swa)
            ga = stage_and_gather(0, ra, sga)

            @pl.when(s > 0)
            def _():
                drain_write(1, swb)
            gb = stage_and_gather(1, rb, sgb)

            for c in ga:
                c.wait()
            pltpu.async_copy(rows_v.at[0], out_hbm.at[pl.ds(ra // 4, _GROUP)], swa)
            for c in gb:
                c.wait()
            pltpu.async_copy(rows_v.at[1], out_hbm.at[pl.ds(rb // 4, _GROUP)], swb)
            return ()

        lax.fori_loop(0, n_iter, body, ())
        drain_write(0, swa)
        drain_write(1, swb)

    return k(T4, keys)


def kernel(x, table, W, b):
    B, L = x.shape
    T4 = _quad_table(table, W, b)
    keys = _quad_keys(x).reshape(-1)
    out = _sc_gather(T4, keys)
    return out.reshape(B, L, _EMB)


# speedup vs baseline: 1.1449x; 1.1420x over previous
import functools

import jax
import jax.numpy as jnp
from jax import lax
from jax.experimental import pallas as pl
from jax.experimental.pallas import tpu as pltpu
from jax.experimental.pallas import tpu_sc as plsc

_VOCAB = 10
_EMB = 20
_QPR = 50
_KROW = 128


def _quad_table_body(table_ref, w_ref, b_ref, t4_ref):
    t = (
        lax.dot_general(
            table_ref[...], w_ref[...],
            dimension_numbers=(((1,), (1,)), ((), ())),
            preferred_element_type=jnp.float32,
            precision=lax.Precision.HIGHEST,
        )
        + b_ref[...]
    )
    v = _VOCAB
    left = jnp.broadcast_to(t[:, None, :], (v, v, _EMB)).reshape(v * v, _EMB)
    right = jnp.broadcast_to(t[None, :, :], (v, v, _EMB)).reshape(v * v, _EMB)
    t2 = jnp.concatenate([left, right], axis=1)
    p = v * v
    left4 = jnp.broadcast_to(t2[:, None, :], (p, p, 2 * _EMB)).reshape(p * p, 2 * _EMB)
    right4 = jnp.broadcast_to(t2[None, :, :], (p, p, 2 * _EMB)).reshape(p * p, 2 * _EMB)
    t4_ref[...] = jnp.concatenate([left4, right4], axis=1)


def _quad_table(table, W, b):
    V, E = table.shape
    return pl.pallas_call(
        _quad_table_body,
        out_shape=jax.ShapeDtypeStruct((V**4, 4 * E), jnp.float32),
    )(table, W, b.reshape(1, E))


def _keys_body(x_ref, k_ref):
    bm, L = x_ref.shape
    xf = x_ref[...].astype(jnp.float32)
    d = lax.broadcasted_iota(jnp.int32, (L, _QPR), 0)
    q = lax.broadcasted_iota(jnp.int32, (L, _QPR), 1)
    pa = jnp.where(d == 4 * q, 10.0, 0.0) + jnp.where(d == 4 * q + 1, 1.0, 0.0)
    pb = jnp.where(d == 4 * q + 2, 10.0, 0.0) + jnp.where(d == 4 * q + 3, 1.0, 0.0)
    ka = lax.dot_general(xf, pa, (((1,), (0,)), ((), ())),
                         preferred_element_type=jnp.float32,
                         precision=lax.Precision.HIGHEST)
    kb = lax.dot_general(xf, pb, (((1,), (0,)), ((), ())),
                         preferred_element_type=jnp.float32,
                         precision=lax.Precision.HIGHEST)
    k = ka.astype(jnp.int32) * 100 + kb.astype(jnp.int32)
    k_ref[...] = jnp.concatenate(
        [k, jnp.zeros((bm, _KROW - _QPR), jnp.int32)], axis=1)


def _quad_keys(x):
    B, L = x.shape
    BM = 512
    return pl.pallas_call(
        _keys_body,
        out_shape=jax.ShapeDtypeStruct((B, _KROW), jnp.int32),
        grid=(B // BM,),
        in_specs=[pl.BlockSpec((BM, L), lambda i: (i, 0))],
        out_specs=pl.BlockSpec((BM, _KROW), lambda i: (i, 0)),
    )(x)


_BQ = 200
_GROUP = 2


_GROWS = 8
_QPAD = 56


def _sc_gather(T4, keys):
    B = keys.shape[0]
    D = T4.shape[1]
    info = plsc.get_sparse_core_info()
    NC, NS = info.num_cores, info.num_subcores
    NW = NC * NS
    rows_per_worker = B // NW
    n_iter = rows_per_worker // (2 * _GROWS)

    mesh = plsc.VectorSubcoreMesh(core_axis_name="c", subcore_axis_name="s")

    @functools.partial(
        pl.kernel,
        out_type=jax.ShapeDtypeStruct((B, _QPR, D), jnp.float32),
        mesh=mesh,
        scratch_types=[
            pltpu.VMEM((_GROWS, _KROW), jnp.int32),
            pltpu.VMEM((_GROWS, _KROW), jnp.int32),
            pltpu.VMEM((2, _GROWS, _QPAD, D), jnp.float32),
            pltpu.SemaphoreType.DMA,
            pltpu.SemaphoreType.DMA,
            pltpu.SemaphoreType.DMA,
            pltpu.SemaphoreType.DMA,
        ],
        compiler_params=pltpu.CompilerParams(
            use_tc_tiling_on_sc=False, needs_layout_passes=False),
    )
    def k(t4_hbm, k_hbm, out_hbm, ka_v, kb_v, rows_v, sga, sgb, swa, swb):
        wid = lax.axis_index("s") * NC + lax.axis_index("c")
        base = wid * rows_per_worker

        def stage_and_gather(g, r0, sem):
            keys_v = (ka_v, kb_v)[g]
            pltpu.sync_copy(k_hbm.at[pl.ds(r0, _GROWS)], keys_v)
            copies = []
            for j in range(_GROWS):
                copies.append(pltpu.async_copy(
                    t4_hbm.at[keys_v.at[j, pl.ds(0, 48)]],
                    rows_v.at[g, j, pl.ds(0, 48)], sem))
                copies.append(pltpu.async_copy(
                    t4_hbm.at[keys_v.at[j, pl.ds(48, 8)]],
                    rows_v.at[g, j, pl.ds(48, 8)], sem))
            return copies

        def drain_write(g, sem):
            pltpu.make_async_copy(
                out_hbm.at[pl.ds(0, _GROWS)],
                rows_v.at[g, :, pl.ds(0, _QPR)], sem).wait()

        def body(s, _):
            ra = base + s * 2 * _GROWS
            rb = ra + _GROWS

            @pl.when(s > 0)
            def _():
                drain_write(0, swa)
            ga = stage_and_gather(0, ra, sga)

            @pl.when(s > 0)
            def _():
                drain_write(1, swb)
            gb = stage_and_gather(1, rb, sgb)

            for c in ga:
                c.wait()
            pltpu.async_copy(rows_v.at[0, :, pl.ds(0, _QPR)],
                             out_hbm.at[pl.ds(ra, _GROWS)], swa)
            for c in gb:
                c.wait()
            pltpu.async_copy(rows_v.at[1, :, pl.ds(0, _QPR)],
                             out_hbm.at[pl.ds(rb, _GROWS)], swb)
            return ()

        lax.fori_loop(0, n_iter, body, ())
        drain_write(0, swa)
        drain_write(1, swb)

    return k(T4, keys)


def kernel(x, table, W, b):
    B, L = x.shape
    T4 = _quad_table(table, W, b)
    keys = _quad_keys(x)
    out = _sc_gather(T4, keys)
    return out.reshape(B, L, _EMB)
